# baseline (speedup 1.0000x reference)
import functools
import math

import jax
import jax.numpy as jnp
from jax import lax
from jax.experimental import pallas as pl
from jax.experimental.pallas import tpu as pltpu
from jax.experimental.pallas import tpu_sc as plsc

D_MODEL = 64
SCALE = math.sqrt(D_MODEL)

V = 1000000
NC = 2
NS = 16
NW = NC * NS

NJ = 200
CH = 128
GP = 136


def _emb_kernel(x_hbm, tab_hbm, out_hbm,
                x_v, p0, p1, g0, g1, o0, o1, gsem0, gsem1, osem0, osem1):
    wid = lax.axis_index("s") * NC + lax.axis_index("c")
    pltpu.sync_copy(x_hbm.at[wid], x_v)

    pbufs, gbufs, obufs = (p0, p1), (g0, g1), (o0, o1)
    gsems, osems = (gsem0, gsem1), (osem0, osem1)

    def compute_pairs(j, b):
        @plsc.parallel_loop(0, CH // 16, unroll=8)
        def pair_blk(l0):
            x16 = x_v[j, pl.ds(l0 * 16, 16)]
            pbufs[b][pl.ds(l0 * 16, 16)] = lax.shift_right_logical(x16, 1)

    def start_gather(j, b):
        pltpu.async_copy(tab_hbm.at[pbufs[b]], gbufs[b].at[:, pl.ds(0, 128)],
                         gsems[b])

    def wait_gather(b):
        pltpu.make_async_copy(tab_hbm.at[pbufs[b]],
                              gbufs[b].at[:, pl.ds(0, 128)], gsems[b]).wait()

    def scale_transpose(j, b):
        row_base = lax.broadcasted_iota(jnp.int32, (16,), 0)

        @plsc.parallel_loop(0, CH // 16, unroll=4)
        def scale_blk(l0):
            x16 = x_v[j, pl.ds(l0 * 16, 16)]
            col0 = lax.mul(lax.bitwise_and(x16, 1), jnp.int32(D_MODEL))
            row16 = row_base + l0 * 16
            for f in range(D_MODEL):
                v = plsc.load_gather(gbufs[b], [row16, col0 + f])
                obufs[b][f // 8, f % 8, pl.ds(l0 * 16, 16)] = v * SCALE

    def start_store(j, b):
        pltpu.async_copy(obufs[b], out_hbm.at[j, :, wid], osems[b])

    def wait_store(b):
        pltpu.make_async_copy(obufs[b], out_hbm.at[0, :, wid], osems[b]).wait()

    for b in range(2):
        compute_pairs(b, b)
        start_gather(b, b)
    for b in range(2):
        wait_gather(b)
        scale_transpose(b, b)
        start_store(b, b)
        compute_pairs(b + 2, b)
        start_gather(b + 2, b)

    def body(i, c):
        for b in range(2):
            j = 2 * i + b
            wait_gather(b)
            wait_store(b)
            scale_transpose(j, b)
            start_store(j, b)

            @pl.when(j + 2 < NJ)
            def _():
                compute_pairs(j + 2, b)
                start_gather(j + 2, b)
        return c

    lax.fori_loop(1, NJ // 2, body, 0)
    wait_store(0)
    wait_store(1)


@jax.jit
def _embedding(xw, tab5):
    mesh = plsc.VectorSubcoreMesh(core_axis_name="c", subcore_axis_name="s")
    kfn = functools.partial(
        pl.kernel,
        mesh=mesh,
        out_type=jax.ShapeDtypeStruct((NJ, 8, NW, 8, 128), jnp.float32),
        scratch_types=[
            pltpu.VMEM((NJ, CH), jnp.int32),
            pltpu.VMEM((CH,), jnp.int32),
            pltpu.VMEM((CH,), jnp.int32),
            pltpu.VMEM((CH, GP), jnp.float32),
            pltpu.VMEM((CH, GP), jnp.float32),
            pltpu.VMEM((8, 8, 128), jnp.float32),
            pltpu.VMEM((8, 8, 128), jnp.float32),
            pltpu.SemaphoreType.DMA,
            pltpu.SemaphoreType.DMA,
            pltpu.SemaphoreType.DMA,
            pltpu.SemaphoreType.DMA,
        ],
        compiler_params=pltpu.CompilerParams(
            use_tc_tiling_on_sc=True, needs_layout_passes=False
        ),
    )(_emb_kernel)
    return kfn(xw, tab5)


def kernel(x, table):
    xw = x.astype(jnp.int32).T.reshape(NJ, NW, CH).transpose(1, 0, 2)
    tab5 = table.reshape(V // 2, 128)
    out5 = _embedding(xw, tab5)
    out = out5.transpose(2, 4, 0, 1, 3).reshape(NW * CH, NJ, D_MODEL)
    return out


# speedup vs baseline: 1.1008x; 1.1008x over previous
import functools
import math

import jax
import jax.numpy as jnp
from jax import lax
from jax.experimental import pallas as pl
from jax.experimental.pallas import tpu as pltpu
from jax.experimental.pallas import tpu_sc as plsc

D_MODEL = 64
SCALE = math.sqrt(D_MODEL)

V = 1000000
NC = 2
NS = 16
NW = NC * NS

NJ = 200
CH = 128
GP = 136


def _emb_kernel(x_hbm, tab_hbm, out_hbm,
                x_v, p0, p1, g0, g1, o0, o1, gsem0, gsem1, osem0, osem1):
    wid = lax.axis_index("s") * NC + lax.axis_index("c")
    pltpu.sync_copy(x_hbm.at[wid], x_v)

    pbufs, gbufs, obufs = (p0, p1), (g0, g1), (o0, o1)
    gsems, osems = (gsem0, gsem1), (osem0, osem1)

    def compute_pairs(j, b):
        @plsc.parallel_loop(0, CH // 16, unroll=8)
        def pair_blk(l0):
            x16 = x_v[j, pl.ds(l0 * 16, 16)]
            pbufs[b][pl.ds(l0 * 16, 16)] = lax.shift_right_logical(x16, 1)

    def start_gather(j, b):
        pltpu.async_copy(tab_hbm.at[pbufs[b]], gbufs[b].at[:, pl.ds(0, 128)],
                         gsems[b])

    def wait_gather(b):
        pltpu.make_async_copy(tab_hbm.at[pbufs[b]],
                              gbufs[b].at[:, pl.ds(0, 128)], gsems[b]).wait()

    def scale_transpose(j, b):
        row_base = lax.broadcasted_iota(jnp.int32, (16,), 0)

        @plsc.parallel_loop(0, CH // 16, unroll=1)
        def scale_blk(l0):
            x16 = x_v[j, pl.ds(l0 * 16, 16)]
            col0 = lax.mul(lax.bitwise_and(x16, 1), jnp.int32(D_MODEL))
            row16 = row_base + l0 * 16
            for f in range(D_MODEL):
                v = plsc.load_gather(gbufs[b], [row16, col0 + f])
                obufs[b][f // 8, f % 8, pl.ds(l0 * 16, 16)] = v * SCALE

    def start_store(j, b):
        pltpu.async_copy(obufs[b], out_hbm.at[j, :, wid], osems[b])

    def wait_store(b):
        pltpu.make_async_copy(obufs[b], out_hbm.at[0, :, wid], osems[b]).wait()

    for b in range(2):
        compute_pairs(b, b)
        start_gather(b, b)
    for b in range(2):
        wait_gather(b)
        scale_transpose(b, b)
        start_store(b, b)
        compute_pairs(b + 2, b)
        start_gather(b + 2, b)

    def body(i, c):
        for b in range(2):
            j = 2 * i + b
            wait_gather(b)
            wait_store(b)
            scale_transpose(j, b)
            start_store(j, b)

            @pl.when(j + 2 < NJ)
            def _():
                compute_pairs(j + 2, b)
                start_gather(j + 2, b)
        return c

    lax.fori_loop(1, NJ // 2, body, 0)
    wait_store(0)
    wait_store(1)


@jax.jit
def _embedding(xw, tab5):
    mesh = plsc.VectorSubcoreMesh(core_axis_name="c", subcore_axis_name="s")
    kfn = functools.partial(
        pl.kernel,
        mesh=mesh,
        out_type=jax.ShapeDtypeStruct((NJ, 8, NW, 8, 128), jnp.float32),
        scratch_types=[
            pltpu.VMEM((NJ, CH), jnp.int32),
            pltpu.VMEM((CH,), jnp.int32),
            pltpu.VMEM((CH,), jnp.int32),
            pltpu.VMEM((CH, GP), jnp.float32),
            pltpu.VMEM((CH, GP), jnp.float32),
            pltpu.VMEM((8, 8, 128), jnp.float32),
            pltpu.VMEM((8, 8, 128), jnp.float32),
            pltpu.SemaphoreType.DMA,
            pltpu.SemaphoreType.DMA,
            pltpu.SemaphoreType.DMA,
            pltpu.SemaphoreType.DMA,
        ],
        compiler_params=pltpu.CompilerParams(
            use_tc_tiling_on_sc=True, needs_layout_passes=False
        ),
    )(_emb_kernel)
    return kfn(xw, tab5)


def kernel(x, table):
    xw = x.astype(jnp.int32).T.reshape(NJ, NW, CH).transpose(1, 0, 2)
    tab5 = table.reshape(V // 2, 128)
    out5 = _embedding(xw, tab5)
    out = out5.transpose(2, 4, 0, 1, 3).reshape(NW * CH, NJ, D_MODEL)
    return out
